# baseline (speedup 1.0000x reference)
import functools

import jax
import jax.numpy as jnp
from jax import lax
from jax.experimental import pallas as pl
from jax.experimental.pallas import tpu as pltpu
from jax.experimental.pallas import tpu_sc as plsc

B = 16384
D = 32
NC = 2
NS = 16
NW = NC * NS
BPW = B // NW
CHUNK = 128
NCHUNK = BPW // CHUNK
GROUPS = BPW // 16


def _sc_body(idx_hbm, ent_hbm, rel_hbm, out_hbm,
             hidx_v, ridx_v, tidx_v, h_v, r_v, t_v, out_v, sem):
    wid = lax.axis_index("s") * NC + lax.axis_index("c")
    base = wid * BPW
    crow = wid * NCHUNK

    pltpu.sync_copy(idx_hbm.at[0, pl.ds(crow, NCHUNK)], hidx_v)
    pltpu.sync_copy(idx_hbm.at[1, pl.ds(crow, NCHUNK)], ridx_v)
    pltpu.sync_copy(idx_hbm.at[2, pl.ds(crow, NCHUNK)], tidx_v)

    copies = []
    for j in range(NCHUNK):
        dst = pl.ds(j * CHUNK, CHUNK)
        copies.append(pltpu.async_copy(ent_hbm.at[hidx_v.at[j]], h_v.at[dst], sem))
        copies.append(pltpu.async_copy(rel_hbm.at[ridx_v.at[j]], r_v.at[dst], sem))
        copies.append(pltpu.async_copy(ent_hbm.at[tidx_v.at[j]], t_v.at[dst], sem))
    for c in copies:
        c.wait()

    lane = lax.iota(jnp.int32, 16)

    def g_body(g, carry):
        row = g * 16 + lane
        acc = jnp.zeros((16,), jnp.float32)
        for d in range(D):
            col = jnp.full((16,), d, jnp.int32)
            hv = plsc.load_gather(h_v, [row, col])
            rv = plsc.load_gather(r_v, [row, col])
            tv = plsc.load_gather(t_v, [row, col])
            acc = acc + hv * rv * tv
        plsc.store_scatter(out_v, [row], acc)
        return carry

    lax.fori_loop(0, GROUPS, g_body, 0)
    pltpu.sync_copy(out_v, out_hbm.at[pl.ds(base, BPW)])


@jax.jit
def _score(idx, entity_embedding, relation_embedding):
    mesh = plsc.VectorSubcoreMesh(core_axis_name="c", subcore_axis_name="s")
    run = pl.kernel(
        _sc_body,
        out_type=jax.ShapeDtypeStruct((B,), jnp.float32),
        mesh=mesh,
        compiler_params=pltpu.CompilerParams(
            needs_layout_passes=False, use_tc_tiling_on_sc=False),
        scratch_types=[
            pltpu.VMEM((NCHUNK, CHUNK), jnp.int32),
            pltpu.VMEM((NCHUNK, CHUNK), jnp.int32),
            pltpu.VMEM((NCHUNK, CHUNK), jnp.int32),
            pltpu.VMEM((BPW, D), jnp.float32),
            pltpu.VMEM((BPW, D), jnp.float32),
            pltpu.VMEM((BPW, D), jnp.float32),
            pltpu.VMEM((BPW,), jnp.float32),
            pltpu.SemaphoreType.DMA,
        ],
    )
    return run(idx, entity_embedding, relation_embedding)


def kernel(sample, entity_embedding, relation_embedding):
    idx = sample.T.reshape(3, B // CHUNK, CHUNK)
    score = _score(idx, entity_embedding, relation_embedding)
    return score.reshape(B, 1)


# speedup vs baseline: 2.3276x; 2.3276x over previous
import functools

import jax
import jax.numpy as jnp
from jax import lax
from jax.experimental import pallas as pl
from jax.experimental.pallas import tpu as pltpu
from jax.experimental.pallas import tpu_sc as plsc

B = 16384
D = 32
NC = 2
NS = 16
NW = NC * NS
BPW = B // NW
GROUPS = BPW // 16


def _sc_body(idx_hbm, ent_hbm, rel_hbm, out_hbm,
             hstage, rstage, tstage, hbuf, rbuf, tbuf, out_v, sem):
    wid = lax.axis_index("s") * NC + lax.axis_index("c")
    base = wid * BPW
    crow = wid * 4

    for c in range(4):
        pltpu.sync_copy(idx_hbm.at[0, crow + c], hstage.at[pl.ds(c * 128, 128)])
        pltpu.sync_copy(idx_hbm.at[1, crow + c], rstage.at[pl.ds(c * 128, 128)])
        pltpu.sync_copy(idx_hbm.at[2, crow + c], tstage.at[pl.ds(c * 128, 128)])

    lane16 = lax.iota(jnp.int32, 16)

    def fire(slot, he, re, te, lane):
        eh = pl.multiple_of((he[lane] >> 7) * 128, 128)
        er = pl.multiple_of((re[lane] >> 7) * 128, 128)
        et = pl.multiple_of((te[lane] >> 7) * 128, 128)
        cps = []
        for dt in range(4):
            dst = pl.ds((slot * 4 + dt) * 8, 8)
            cps.append(pltpu.async_copy(
                ent_hbm.at[dt, :, pl.ds(eh, 128)], hbuf.at[dst], sem))
            cps.append(pltpu.async_copy(
                rel_hbm.at[dt, :, pl.ds(er, 128)], rbuf.at[dst], sem))
            cps.append(pltpu.async_copy(
                ent_hbm.at[dt, :, pl.ds(et, 128)], tbuf.at[dst], sem))
        return cps

    def score(slot, he, re, te, lane, opos):
        rows_a = slot * 32 + lane16
        rows_b = rows_a + 16
        lh = jnp.full((16,), he[lane] & 127, jnp.int32)
        lr = jnp.full((16,), re[lane] & 127, jnp.int32)
        lt = jnp.full((16,), te[lane] & 127, jnp.int32)
        pa = (plsc.load_gather(hbuf, [rows_a, lh])
              * plsc.load_gather(rbuf, [rows_a, lr])
              * plsc.load_gather(tbuf, [rows_a, lt]))
        pb = (plsc.load_gather(hbuf, [rows_b, lh])
              * plsc.load_gather(rbuf, [rows_b, lr])
              * plsc.load_gather(tbuf, [rows_b, lt]))
        s = jnp.sum(pa + pb)
        plsc.store_scatter(out_v, [jnp.full((16,), opos, jnp.int32)],
                           jnp.full((16,), s, jnp.float32),
                           mask=lane16 == 0)

    def g_body(g, carry):
        off = g * 16
        he = hstage[pl.ds(off, 16)]
        re = rstage[pl.ds(off, 16)]
        te = tstage[pl.ds(off, 16)]
        cps = fire(0, he, re, te, 0)
        for lane in range(16):
            nxt = None
            if lane + 1 < 16:
                nxt = fire((lane + 1) % 2, he, re, te, lane + 1)
            for cp in cps:
                cp.wait()
            score(lane % 2, he, re, te, lane, off + lane)
            cps = nxt
        return carry

    lax.fori_loop(0, GROUPS, g_body, 0)
    pltpu.sync_copy(out_v, out_hbm.at[pl.ds(base, BPW)])


@jax.jit
def _score(idx, ent4, rel4):
    mesh = plsc.VectorSubcoreMesh(core_axis_name="c", subcore_axis_name="s")
    run = pl.kernel(
        _sc_body,
        out_type=jax.ShapeDtypeStruct((B,), jnp.float32),
        mesh=mesh,
        compiler_params=pltpu.CompilerParams(needs_layout_passes=False),
        scratch_types=[
            pltpu.VMEM((BPW,), jnp.int32),
            pltpu.VMEM((BPW,), jnp.int32),
            pltpu.VMEM((BPW,), jnp.int32),
            pltpu.VMEM((64, 128), jnp.float32),
            pltpu.VMEM((64, 128), jnp.float32),
            pltpu.VMEM((64, 128), jnp.float32),
            pltpu.VMEM((BPW,), jnp.float32),
            pltpu.SemaphoreType.DMA,
        ],
    )
    return run(idx, ent4, rel4)


def kernel(sample, entity_embedding, relation_embedding):
    idx = sample.T.reshape(3, B // 128, 128)
    ent4 = entity_embedding.T.reshape(4, 8, 1000000)
    rel4 = relation_embedding.T.reshape(4, 8, 1000000)
    score = _score(idx, ent4, rel4)
    return score.reshape(B, 1)


# speedup vs baseline: 2.3697x; 1.0181x over previous
import functools

import jax
import jax.numpy as jnp
from jax import lax
from jax.experimental import pallas as pl
from jax.experimental.pallas import tpu as pltpu
from jax.experimental.pallas import tpu_sc as plsc

B = 16384
D = 32
NC = 2
NS = 16
NW = NC * NS
BPW = B // NW
GROUPS = BPW // 16


def _sc_body(idx_hbm, ent_hbm, rel_hbm, out_hbm,
             hstage, rstage, tstage, hbuf, rbuf, tbuf, out_v, sem):
    wid = lax.axis_index("s") * NC + lax.axis_index("c")
    base = wid * BPW
    crow = wid * 4

    for c in range(4):
        pltpu.sync_copy(idx_hbm.at[0, crow + c], hstage.at[pl.ds(c * 128, 128)])
        pltpu.sync_copy(idx_hbm.at[1, crow + c], rstage.at[pl.ds(c * 128, 128)])
        pltpu.sync_copy(idx_hbm.at[2, crow + c], tstage.at[pl.ds(c * 128, 128)])

    lane16 = lax.iota(jnp.int32, 16)

    def fire(slot, he, re, te, lane):
        eh = pl.multiple_of((he[lane] >> 7) * 128, 128)
        er = pl.multiple_of((re[lane] >> 7) * 128, 128)
        et = pl.multiple_of((te[lane] >> 7) * 128, 128)
        dst = pl.ds(slot * 4, 4)
        return [
            pltpu.async_copy(ent_hbm.at[:, :, pl.ds(eh, 128)],
                             hbuf.at[dst], sem),
            pltpu.async_copy(rel_hbm.at[:, :, pl.ds(er, 128)],
                             rbuf.at[dst], sem),
            pltpu.async_copy(ent_hbm.at[:, :, pl.ds(et, 128)],
                             tbuf.at[dst], sem),
        ]

    def score(slot, he, re, te, lane, opos):
        dts_a = jnp.full((16,), slot * 4, jnp.int32) + (lane16 >> 3)
        dts_b = dts_a + 2
        dss = lane16 & 7
        lh = jnp.full((16,), he[lane] & 127, jnp.int32)
        lr = jnp.full((16,), re[lane] & 127, jnp.int32)
        lt = jnp.full((16,), te[lane] & 127, jnp.int32)
        pa = (plsc.load_gather(hbuf, [dts_a, dss, lh])
              * plsc.load_gather(rbuf, [dts_a, dss, lr])
              * plsc.load_gather(tbuf, [dts_a, dss, lt]))
        pb = (plsc.load_gather(hbuf, [dts_b, dss, lh])
              * plsc.load_gather(rbuf, [dts_b, dss, lr])
              * plsc.load_gather(tbuf, [dts_b, dss, lt]))
        s = jnp.sum(pa + pb)
        plsc.store_scatter(out_v, [jnp.full((16,), opos, jnp.int32)],
                           jnp.full((16,), s, jnp.float32),
                           mask=lane16 == 0)

    def g_body(g, carry):
        off = g * 16
        he = hstage[pl.ds(off, 16)]
        re = rstage[pl.ds(off, 16)]
        te = tstage[pl.ds(off, 16)]
        cps = fire(0, he, re, te, 0)
        for lane in range(16):
            nxt = None
            if lane + 1 < 16:
                nxt = fire((lane + 1) % 2, he, re, te, lane + 1)
            for cp in cps:
                cp.wait()
            score(lane % 2, he, re, te, lane, off + lane)
            cps = nxt
        return carry

    lax.fori_loop(0, GROUPS, g_body, 0)
    pltpu.sync_copy(out_v, out_hbm.at[pl.ds(base, BPW)])


@jax.jit
def _score(idx, ent4, rel4):
    mesh = plsc.VectorSubcoreMesh(core_axis_name="c", subcore_axis_name="s")
    run = pl.kernel(
        _sc_body,
        out_type=jax.ShapeDtypeStruct((B,), jnp.float32),
        mesh=mesh,
        compiler_params=pltpu.CompilerParams(needs_layout_passes=False),
        scratch_types=[
            pltpu.VMEM((BPW,), jnp.int32),
            pltpu.VMEM((BPW,), jnp.int32),
            pltpu.VMEM((BPW,), jnp.int32),
            pltpu.VMEM((8, 8, 128), jnp.float32),
            pltpu.VMEM((8, 8, 128), jnp.float32),
            pltpu.VMEM((8, 8, 128), jnp.float32),
            pltpu.VMEM((BPW,), jnp.float32),
            pltpu.SemaphoreType.DMA,
        ],
    )
    return run(idx, ent4, rel4)


def kernel(sample, entity_embedding, relation_embedding):
    idx = sample.T.reshape(3, B // 128, 128)
    ent4 = entity_embedding.T.reshape(4, 8, 1000000)
    rel4 = relation_embedding.T.reshape(4, 8, 1000000)
    score = _score(idx, ent4, rel4)
    return score.reshape(B, 1)


# speedup vs baseline: 2.8772x; 1.2142x over previous
import functools

import jax
import jax.numpy as jnp
from jax import lax
from jax.experimental import pallas as pl
from jax.experimental.pallas import tpu as pltpu
from jax.experimental.pallas import tpu_sc as plsc

B = 16384
D = 32
NC = 2
NS = 16
NW = NC * NS
BPW = B // NW
GROUPS = BPW // 16


def _sc_body(idx_hbm, ent_hbm, rel_hbm, out_hbm,
             hstage, rstage, tstage, hbuf, rbuf, tbuf, out_v, sem):
    wid = lax.axis_index("s") * NC + lax.axis_index("c")
    base = wid * BPW
    crow = wid * 4

    for c in range(4):
        pltpu.sync_copy(idx_hbm.at[0, crow + c], hstage.at[pl.ds(c * 128, 128)])
        pltpu.sync_copy(idx_hbm.at[1, crow + c], rstage.at[pl.ds(c * 128, 128)])
        pltpu.sync_copy(idx_hbm.at[2, crow + c], tstage.at[pl.ds(c * 128, 128)])

    lane16 = lax.iota(jnp.int32, 16)

    def fire(slot, he, re, te, lane):
        eh = pl.multiple_of((he[lane] >> 7) * 128, 128)
        er = pl.multiple_of((re[lane] >> 7) * 128, 128)
        et = pl.multiple_of((te[lane] >> 7) * 128, 128)
        dst = pl.ds(slot * 4, 4)
        return [
            pltpu.async_copy(ent_hbm.at[:, :, pl.ds(eh, 128)],
                             hbuf.at[dst], sem),
            pltpu.async_copy(rel_hbm.at[:, :, pl.ds(er, 128)],
                             rbuf.at[dst], sem),
            pltpu.async_copy(ent_hbm.at[:, :, pl.ds(et, 128)],
                             tbuf.at[dst], sem),
        ]

    def score(slot, he, re, te, lane, opos):
        dts_a = jnp.full((16,), slot * 4, jnp.int32) + (lane16 >> 3)
        dts_b = dts_a + 2
        dss = lane16 & 7
        lh = jnp.full((16,), he[lane] & 127, jnp.int32)
        lr = jnp.full((16,), re[lane] & 127, jnp.int32)
        lt = jnp.full((16,), te[lane] & 127, jnp.int32)
        pa = (plsc.load_gather(hbuf, [dts_a, dss, lh])
              * plsc.load_gather(rbuf, [dts_a, dss, lr])
              * plsc.load_gather(tbuf, [dts_a, dss, lt]))
        pb = (plsc.load_gather(hbuf, [dts_b, dss, lh])
              * plsc.load_gather(rbuf, [dts_b, dss, lr])
              * plsc.load_gather(tbuf, [dts_b, dss, lt]))
        s = jnp.sum(pa + pb)
        plsc.store_scatter(out_v, [jnp.full((16,), opos, jnp.int32)],
                           jnp.full((16,), s, jnp.float32),
                           mask=lane16 == 0)

    def g_body(g, carry):
        off = g * 16
        he = hstage[pl.ds(off, 16)]
        re = rstage[pl.ds(off, 16)]
        te = tstage[pl.ds(off, 16)]
        cps0 = fire(0, he, re, te, 0)
        cps1 = fire(1, he, re, te, 1)
        for lane in range(16):
            nxt = None
            if lane + 2 < 16:
                nxt = fire((lane + 2) % 4, he, re, te, lane + 2)
            for cp in cps0:
                cp.wait()
            score(lane % 4, he, re, te, lane, off + lane)
            cps0, cps1 = cps1, nxt
        return carry

    lax.fori_loop(0, GROUPS, g_body, 0)
    pltpu.sync_copy(out_v, out_hbm.at[pl.ds(base, BPW)])


@jax.jit
def _score(idx, ent4, rel4):
    mesh = plsc.VectorSubcoreMesh(core_axis_name="c", subcore_axis_name="s")
    run = pl.kernel(
        _sc_body,
        out_type=jax.ShapeDtypeStruct((B,), jnp.float32),
        mesh=mesh,
        compiler_params=pltpu.CompilerParams(needs_layout_passes=False),
        scratch_types=[
            pltpu.VMEM((BPW,), jnp.int32),
            pltpu.VMEM((BPW,), jnp.int32),
            pltpu.VMEM((BPW,), jnp.int32),
            pltpu.VMEM((16, 8, 128), jnp.float32),
            pltpu.VMEM((16, 8, 128), jnp.float32),
            pltpu.VMEM((16, 8, 128), jnp.float32),
            pltpu.VMEM((BPW,), jnp.float32),
            pltpu.SemaphoreType.DMA,
        ],
    )
    return run(idx, ent4, rel4)


def kernel(sample, entity_embedding, relation_embedding):
    idx = sample.T.reshape(3, B // 128, 128)
    ent4 = entity_embedding.T.reshape(4, 8, 1000000)
    rel4 = relation_embedding.T.reshape(4, 8, 1000000)
    score = _score(idx, ent4, rel4)
    return score.reshape(B, 1)


# speedup vs baseline: 2.9497x; 1.0252x over previous
import functools

import jax
import jax.numpy as jnp
from jax import lax
from jax.experimental import pallas as pl
from jax.experimental.pallas import tpu as pltpu
from jax.experimental.pallas import tpu_sc as plsc

B = 16384
D = 32
NC = 2
NS = 16
NW = NC * NS
BPW = B // NW
GROUPS = BPW // 16


def _sc_body(idx_hbm, ent_hbm, rel_hbm, out_hbm,
             hstage, rstage, tstage, hbuf, rbuf, tbuf, out_v, sem):
    wid = lax.axis_index("s") * NC + lax.axis_index("c")
    base = wid * BPW
    crow = wid * 4

    for c in range(4):
        pltpu.sync_copy(idx_hbm.at[0, crow + c], hstage.at[pl.ds(c * 128, 128)])
        pltpu.sync_copy(idx_hbm.at[1, crow + c], rstage.at[pl.ds(c * 128, 128)])
        pltpu.sync_copy(idx_hbm.at[2, crow + c], tstage.at[pl.ds(c * 128, 128)])

    lane16 = lax.iota(jnp.int32, 16)

    def fire(slot, he, re, te, lane):
        eh = pl.multiple_of((he[lane] >> 7) * 128, 128)
        er = pl.multiple_of((re[lane] >> 7) * 128, 128)
        et = pl.multiple_of((te[lane] >> 7) * 128, 128)
        dst = pl.ds(slot * 4, 4)
        return [
            pltpu.async_copy(ent_hbm.at[:, :, pl.ds(eh, 128)],
                             hbuf.at[dst], sem),
            pltpu.async_copy(rel_hbm.at[:, :, pl.ds(er, 128)],
                             rbuf.at[dst], sem),
            pltpu.async_copy(ent_hbm.at[:, :, pl.ds(et, 128)],
                             tbuf.at[dst], sem),
        ]

    def score(slot, he, re, te, lane, opos):
        dts_a = jnp.full((16,), slot * 4, jnp.int32) + (lane16 >> 3)
        dts_b = dts_a + 2
        dss = lane16 & 7
        lh = jnp.full((16,), he[lane] & 127, jnp.int32)
        lr = jnp.full((16,), re[lane] & 127, jnp.int32)
        lt = jnp.full((16,), te[lane] & 127, jnp.int32)
        pa = (plsc.load_gather(hbuf, [dts_a, dss, lh])
              * plsc.load_gather(rbuf, [dts_a, dss, lr])
              * plsc.load_gather(tbuf, [dts_a, dss, lt]))
        pb = (plsc.load_gather(hbuf, [dts_b, dss, lh])
              * plsc.load_gather(rbuf, [dts_b, dss, lr])
              * plsc.load_gather(tbuf, [dts_b, dss, lt]))
        s = jnp.sum(pa + pb)
        plsc.store_scatter(out_v, [jnp.full((16,), opos, jnp.int32)],
                           jnp.full((16,), s, jnp.float32),
                           mask=lane16 == 0)

    def g_body(g, carry):
        off = g * 16
        he = hstage[pl.ds(off, 16)]
        re = rstage[pl.ds(off, 16)]
        te = tstage[pl.ds(off, 16)]
        depth = 4
        inflight = [fire(i, he, re, te, i) for i in range(depth)]
        for lane in range(16):
            nxt = None
            if lane + depth < 16:
                nxt = fire((lane + depth) % 8, he, re, te, lane + depth)
            for cp in inflight[0]:
                cp.wait()
            score(lane % 8, he, re, te, lane, off + lane)
            inflight = inflight[1:] + [nxt]
        return carry

    lax.fori_loop(0, GROUPS, g_body, 0)
    pltpu.sync_copy(out_v, out_hbm.at[pl.ds(base, BPW)])


@jax.jit
def _score(idx, ent4, rel4):
    mesh = plsc.VectorSubcoreMesh(core_axis_name="c", subcore_axis_name="s")
    run = pl.kernel(
        _sc_body,
        out_type=jax.ShapeDtypeStruct((B,), jnp.float32),
        mesh=mesh,
        compiler_params=pltpu.CompilerParams(needs_layout_passes=False),
        scratch_types=[
            pltpu.VMEM((BPW,), jnp.int32),
            pltpu.VMEM((BPW,), jnp.int32),
            pltpu.VMEM((BPW,), jnp.int32),
            pltpu.VMEM((32, 8, 128), jnp.float32),
            pltpu.VMEM((32, 8, 128), jnp.float32),
            pltpu.VMEM((32, 8, 128), jnp.float32),
            pltpu.VMEM((BPW,), jnp.float32),
            pltpu.SemaphoreType.DMA,
        ],
    )
    return run(idx, ent4, rel4)


def kernel(sample, entity_embedding, relation_embedding):
    idx = sample.T.reshape(3, B // 128, 128)
    ent4 = entity_embedding.T.reshape(4, 8, 1000000)
    rel4 = relation_embedding.T.reshape(4, 8, 1000000)
    score = _score(idx, ent4, rel4)
    return score.reshape(B, 1)
